# baseline (speedup 1.0000x reference)
import functools

import jax
import jax.numpy as jnp
from jax import lax
from jax.experimental import pallas as pl
from jax.experimental.pallas import tpu as pltpu
from jax.experimental.pallas import tpu_sc as plsc

_VOCAB = 50257
_HIDDEN = 1024
_B = 4
_S = 2048
_N = _B * _S
_NC = 2
_NS = 16
_NW = _NC * _NS
_TPW = _N // _NW
_C = 16
_NCH = _TPW // _C
_LANES = 16


def _emb_body(ids_hbm, word_hbm, pos_hbm, out_hbm, idx_v, rows_v, pos_v,
              gsem, psem):
  wid = lax.axis_index("s") * _NC + lax.axis_index("c")
  base = wid * _TPW
  pos_base = lax.rem(base, _S)

  pltpu.sync_copy(ids_hbm.at[wid], idx_v)

  def chunk_body(ch, carry):
    g = pltpu.async_copy(word_hbm.at[idx_v.at[ch]], rows_v, gsem)
    p = pltpu.async_copy(pos_hbm.at[pl.ds(pos_base + ch * _C, _C)], pos_v,
                         psem)
    g.wait()
    p.wait()

    def row_body(r, c2):
      for grp in range(_HIDDEN // _LANES):
        sl = pl.ds(grp * _LANES, _LANES)
        rows_v[r, sl] += pos_v[r, sl]
      return c2

    lax.fori_loop(0, _C, row_body, 0)
    pltpu.sync_copy(rows_v, out_hbm.at[pl.ds(base + ch * _C, _C)])
    return carry

  lax.fori_loop(0, _NCH, chunk_body, 0)


_mesh = plsc.VectorSubcoreMesh(
    core_axis_name="c", subcore_axis_name="s", num_cores=_NC,
    num_subcores=_NS)

_emb_kernel = functools.partial(
    pl.kernel,
    out_type=jax.ShapeDtypeStruct((_N, _HIDDEN), jnp.float32),
    mesh=_mesh,
    scratch_types=[
        pltpu.VMEM((_NCH, _C), jnp.int32),
        pltpu.VMEM((_C, _HIDDEN), jnp.float32),
        pltpu.VMEM((_C, _HIDDEN), jnp.float32),
        pltpu.SemaphoreType.DMA,
        pltpu.SemaphoreType.DMA,
    ],
)(_emb_body)


@jax.jit
def kernel(input_ids, word_embeddings, position_embeddings):
  ids = input_ids.reshape(_NW, _NCH, _C)
  out = _emb_kernel(ids, word_embeddings, position_embeddings)
  return out.reshape(_B, _S, _HIDDEN)


# speedup vs baseline: 1.1428x; 1.1428x over previous
import functools

import jax
import jax.numpy as jnp
from jax import lax
from jax.experimental import pallas as pl
from jax.experimental.pallas import tpu as pltpu
from jax.experimental.pallas import tpu_sc as plsc

_VOCAB = 50257
_HIDDEN = 1024
_B = 4
_S = 2048
_N = _B * _S
_NC = 2
_NS = 16
_NW = _NC * _NS
_P = _S // _NW
_C = 16
_JC = _P // _C
_NCH = _B * _JC
_LANES = 16


def _emb_body(ids_hbm, word_hbm, pos_hbm, out_hbm, idx_v, rows_v, pos_v,
              gsem0, gsem1, osem0, osem1, psem):
  gsems = (gsem0, gsem1)
  osems = (osem0, osem1)
  wid = lax.axis_index("s") * _NC + lax.axis_index("c")
  pos_base = wid * _P

  pltpu.sync_copy(ids_hbm.at[wid], idx_v)
  pload = pltpu.async_copy(pos_hbm.at[pl.ds(pos_base, _P)], pos_v, psem)

  gathers = [None, None]
  stores = [None, None]
  gathers[0] = pltpu.async_copy(word_hbm.at[idx_v.at[0]], rows_v.at[0],
                                gsems[0])
  pload.wait()

  for ch in range(_NCH):
    cur = ch % 2
    nxt = 1 - cur
    if ch + 1 < _NCH:
      if stores[nxt] is not None:
        stores[nxt].wait()
      gathers[nxt] = pltpu.async_copy(word_hbm.at[idx_v.at[ch + 1]],
                                      rows_v.at[nxt], gsems[nxt])
    gathers[cur].wait()

    b, j = divmod(ch, _JC)

    def row_body(r, c2):
      for grp in range(_HIDDEN // _LANES):
        sl = pl.ds(grp * _LANES, _LANES)
        rows_v[cur, r, sl] += pos_v[j * _C + r, sl]
      return c2

    lax.fori_loop(0, _C, row_body, 0)

    out_off = b * _S + pos_base + j * _C
    stores[cur] = pltpu.async_copy(rows_v.at[cur],
                                   out_hbm.at[pl.ds(out_off, _C)],
                                   osems[cur])

  stores[0].wait()
  stores[1].wait()


_mesh = plsc.VectorSubcoreMesh(
    core_axis_name="c", subcore_axis_name="s", num_cores=_NC,
    num_subcores=_NS)

_emb_kernel = functools.partial(
    pl.kernel,
    out_type=jax.ShapeDtypeStruct((_N, _HIDDEN), jnp.float32),
    mesh=_mesh,
    scratch_types=[
        pltpu.VMEM((_NCH, _C), jnp.int32),
        pltpu.VMEM((2, _C, _HIDDEN), jnp.float32),
        pltpu.VMEM((_P, _HIDDEN), jnp.float32),
        pltpu.SemaphoreType.DMA,
        pltpu.SemaphoreType.DMA,
        pltpu.SemaphoreType.DMA,
        pltpu.SemaphoreType.DMA,
        pltpu.SemaphoreType.DMA,
    ],
)(_emb_body)


@jax.jit
def kernel(input_ids, word_embeddings, position_embeddings):
  ids2 = (input_ids.reshape(_B, _NW, _JC, _C)
          .transpose(1, 0, 2, 3)
          .reshape(_NW, _NCH, _C))
  out = _emb_kernel(ids2, word_embeddings, position_embeddings)
  return out.reshape(_B, _S, _HIDDEN)
